# baseline (speedup 1.0000x reference)
import functools

import jax
import jax.numpy as jnp
from jax import lax
from jax.experimental import pallas as pl
from jax.experimental.pallas import tpu as pltpu
from jax.experimental.pallas import tpu_sc as plsc

N = 10000
D = 256
G = 64
NC = 2
NS = 16
NW = NC * NS
L = 16
ROWS_W = 320
CHUNK = 64
NCHUNK = ROWS_W // CHUNK
SROWS = 80
ZROWS = SROWS // NS
TAIL_W = NW - 1
TAIL_ROWS = N - TAIL_W * ROWS_W
TAIL_CHUNKS = TAIL_ROWS // CHUNK
TAIL_REM = TAIL_ROWS - TAIL_CHUNKS * CHUNK


def _x_copy(x_hbm, xbuf, sem, base, j, nrows):
    return pltpu.make_async_copy(
        x_hbm.at[pl.ds(base + j * CHUNK, nrows)],
        xbuf.at[pl.ds(j * CHUNK, nrows)], sem)


def _sc_body(x_hbm, b_hbm, xsum_hbm, cnt_hbm,
             xbuf, idxbuf, ones, zx, zc, ssum, scnt, *sems):
    c = lax.axis_index("c")
    s = lax.axis_index("s")
    w = c * NS + s
    base = w * ROWS_W

    @pl.when(w < TAIL_W)
    def _():
        for j in range(NCHUNK):
            _x_copy(x_hbm, xbuf, sems[j], base, j, CHUNK).start()

    @pl.when(w == TAIL_W)
    def _():
        for j in range(TAIL_CHUNKS):
            _x_copy(x_hbm, xbuf, sems[j], base, j, CHUNK).start()
        if TAIL_REM:
            _x_copy(x_hbm, xbuf, sems[TAIL_CHUNKS], base, TAIL_CHUNKS,
                    TAIL_REM).start()

    gvec = jnp.full((L,), G, jnp.int32)
    for j in range(NCHUNK):
        for k in range(CHUNK // L):
            idxbuf[j, pl.ds(k * L, L)] = gvec

    @pl.when(w < TAIL_W)
    def _():
        for j in range(NCHUNK):
            pltpu.sync_copy(b_hbm.at[pl.ds(base + j * CHUNK, CHUNK)],
                            idxbuf.at[j])

    @pl.when(w == TAIL_W)
    def _():
        for j in range(TAIL_CHUNKS):
            pltpu.sync_copy(b_hbm.at[pl.ds(base + j * CHUNK, CHUNK)],
                            idxbuf.at[j])
        if TAIL_REM:
            pltpu.sync_copy(
                b_hbm.at[pl.ds(base + TAIL_CHUNKS * CHUNK, TAIL_REM)],
                idxbuf.at[TAIL_CHUNKS].at[pl.ds(0, TAIL_REM)])

    zvec = jnp.zeros((L,), jnp.float32)
    for r in range(ZROWS):
        for k in range(D // L):
            zx[r, pl.ds(k * L, L)] = zvec
        zc[r] = zvec
    onev = jnp.ones((L,), jnp.float32)
    for r in range(CHUNK):
        ones[r] = onev

    pltpu.sync_copy(zx, ssum.at[pl.ds(s * ZROWS, ZROWS)])
    pltpu.sync_copy(zc, scnt.at[pl.ds(s * ZROWS, ZROWS)])

    plsc.subcore_barrier()

    for j in range(NCHUNK):
        @pl.when(w < TAIL_W)
        def _():
            _x_copy(x_hbm, xbuf, sems[j], base, j, CHUNK).wait()

        if j < TAIL_CHUNKS:
            @pl.when(w == TAIL_W)
            def _():
                _x_copy(x_hbm, xbuf, sems[j], base, j, CHUNK).wait()
        elif j == TAIL_CHUNKS and TAIL_REM:
            @pl.when(w == TAIL_W)
            def _():
                _x_copy(x_hbm, xbuf, sems[j], base, j, TAIL_REM).wait()

        pltpu.sync_copy(xbuf.at[pl.ds(j * CHUNK, CHUNK)],
                        ssum.at[idxbuf.at[j]], add=True)
        pltpu.sync_copy(ones, scnt.at[idxbuf.at[j]], add=True)

    plsc.subcore_barrier()

    gpt = G // NS
    pltpu.sync_copy(ssum.at[pl.ds(s * gpt, gpt)],
                    xsum_hbm.at[c].at[pl.ds(s * gpt, gpt)])
    pltpu.sync_copy(scnt.at[pl.ds(s * gpt, gpt)],
                    cnt_hbm.at[c].at[pl.ds(s * gpt, gpt)])


@jax.jit
def _sc_segment(x, b):
    mesh = plsc.VectorSubcoreMesh(core_axis_name="c", subcore_axis_name="s",
                                  num_cores=NC, num_subcores=NS)
    return pl.kernel(
        _sc_body,
        out_type=(jax.ShapeDtypeStruct((NC, G, D), jnp.float32),
                  jax.ShapeDtypeStruct((NC, G, L), jnp.float32)),
        mesh=mesh,
        compiler_params=pltpu.CompilerParams(use_tc_tiling_on_sc=False),
        scratch_types=[
            pltpu.VMEM((ROWS_W, D), jnp.float32),
            pltpu.VMEM((NCHUNK, CHUNK), jnp.int32),
            pltpu.VMEM((CHUNK, L), jnp.float32),
            pltpu.VMEM((ZROWS, D), jnp.float32),
            pltpu.VMEM((ZROWS, L), jnp.float32),
            pltpu.VMEM_SHARED((SROWS, D), jnp.float32),
            pltpu.VMEM_SHARED((SROWS, L), jnp.float32),
        ] + [pltpu.SemaphoreType.DMA] * NCHUNK,
    )(x, b)


def _tc_combine_body(xsum_ref, cnt_ref, u_ref, out_ref):
    ssum = xsum_ref[0] + xsum_ref[1]
    cnt = cnt_ref[0] + cnt_ref[1]
    tot = jnp.sum(ssum, axis=1)
    counts = jnp.sum(cnt, axis=1) * (1.0 / L)
    usum = jnp.sum(u_ref[...], axis=1)
    out_ref[...] = usum + tot / jnp.maximum(counts, 1.0)


@jax.jit
def _tc_combine(xsum, cnt, u):
    return pl.pallas_call(
        _tc_combine_body,
        out_shape=jax.ShapeDtypeStruct((G,), jnp.float32),
    )(xsum, cnt, u)


def kernel(x, edge_index, edge_attr, u, batch):
    del edge_index, edge_attr
    xsum, cnt = _sc_segment(x, batch.astype(jnp.int32))
    return _tc_combine(xsum, cnt, u)


# speedup vs baseline: 1.2216x; 1.2216x over previous
import functools

import jax
import jax.numpy as jnp
from jax import lax
from jax.experimental import pallas as pl
from jax.experimental.pallas import tpu as pltpu
from jax.experimental.pallas import tpu_sc as plsc

N = 10000
D = 256
G = 64
NC = 2
NS = 16
NW = NC * NS
L = 16
ROWS_W = 320
CH = 128
NCHUNK = 3
VROWS = NCHUNK * CH
GROUPS_CH = CH // L
SROWS = 160
TAIL_W = NW - 1
TAIL_ROWS = N - TAIL_W * ROWS_W
FULL_SIZES = (CH, CH, ROWS_W - 2 * CH)
TAIL_SIZES = (TAIL_ROWS, 0, 0)


def _x_copy(x_hbm, xbuf, sem, base, j, nrows):
    return pltpu.make_async_copy(
        x_hbm.at[pl.ds(base + j * CH, nrows)],
        xbuf.at[pl.ds(j * CH, nrows)], sem)


def _sc_body(x_hbm, b_hbm, xsum_hbm, xbuf, bbuf, table, *sems):
    c = lax.axis_index("c")
    s = lax.axis_index("s")
    w = c * NS + s
    base = w * ROWS_W

    @pl.when(w < TAIL_W)
    def _():
        for j, n in enumerate(FULL_SIZES):
            _x_copy(x_hbm, xbuf, sems[j], base, j, n).start()

    @pl.when(w == TAIL_W)
    def _():
        for j, n in enumerate(TAIL_SIZES):
            if n:
                _x_copy(x_hbm, xbuf, sems[j], base, j, n).start()

    gvec = jnp.full((L,), G, jnp.int32)
    for k in range(VROWS // L):
        bbuf[pl.ds(k * L, L)] = gvec

    @pl.when(w < TAIL_W)
    def _():
        pltpu.sync_copy(b_hbm.at[pl.ds(base, ROWS_W)],
                        bbuf.at[pl.ds(0, ROWS_W)])

    @pl.when(w == TAIL_W)
    def _():
        pltpu.sync_copy(b_hbm.at[pl.ds(base, TAIL_ROWS)],
                        bbuf.at[pl.ds(0, TAIL_ROWS)])

    zvec = jnp.zeros((L,), jnp.float32)
    for r in range(SROWS):
        table[r, pl.ds(0, L)] = zvec

    onev = jnp.ones((L,), jnp.float32)

    def group_body(q, _):
        bq = bbuf[pl.ds(q * L, L)]
        for i in range(L):
            r = q * L + i
            acc = xbuf[r, pl.ds(0, L)]
            for k in range(1, D // L):
                acc = acc + xbuf[r, pl.ds(k * L, L)]
            b = bq[i]
            table[b, pl.ds(0, L)] = table[b, pl.ds(0, L)] + acc
            bc = b + (SROWS // 2)
            table[bc, pl.ds(0, L)] = table[bc, pl.ds(0, L)] + onev
        return 0

    for j in range(NCHUNK):
        @pl.when(w < TAIL_W)
        def _():
            _x_copy(x_hbm, xbuf, sems[j], base, j, FULL_SIZES[j]).wait()

        if TAIL_SIZES[j]:
            @pl.when(w == TAIL_W)
            def _():
                _x_copy(x_hbm, xbuf, sems[j], base, j, TAIL_SIZES[j]).wait()

        lax.fori_loop(j * GROUPS_CH, (j + 1) * GROUPS_CH, group_body, 0)

    pltpu.sync_copy(table, xsum_hbm.at[c].at[s])


@jax.jit
def _sc_segment(x, b):
    mesh = plsc.VectorSubcoreMesh(core_axis_name="c", subcore_axis_name="s",
                                  num_cores=NC, num_subcores=NS)
    return pl.kernel(
        _sc_body,
        out_type=jax.ShapeDtypeStruct((NC, NS, SROWS, 128), jnp.float32),
        mesh=mesh,
        scratch_types=[
            pltpu.VMEM((VROWS, D), jnp.float32),
            pltpu.VMEM((VROWS,), jnp.int32),
            pltpu.VMEM((SROWS, 128), jnp.float32),
        ] + [pltpu.SemaphoreType.DMA] * NCHUNK,
    )(x, b)


def _tc_combine_body(xsum_ref, u_ref, out_ref):
    t = jnp.sum(xsum_ref[...], axis=(0, 1))
    tot = jnp.sum(t[:G, :L], axis=1)
    counts = jnp.sum(t[SROWS // 2:SROWS // 2 + G, :L], axis=1) * (1.0 / L)
    usum = jnp.sum(u_ref[...], axis=1)
    out_ref[...] = usum + tot / jnp.maximum(counts, 1.0)


@jax.jit
def _tc_combine(xsum, u):
    return pl.pallas_call(
        _tc_combine_body,
        out_shape=jax.ShapeDtypeStruct((G,), jnp.float32),
    )(xsum, u)


def kernel(x, edge_index, edge_attr, u, batch):
    del edge_index, edge_attr
    b = batch.astype(jnp.int32)
    xsum = _sc_segment(x, b)
    return _tc_combine(xsum, u)
